# Initial kernel scaffold; baseline (speedup 1.0000x reference)
import jax
import jax.numpy as jnp
from jax.experimental import pallas as pl


def kernel(x, Wr, br, W1, b1, W2, b2):
    raise NotImplementedError("write your pallas kernel here")


# speedup vs baseline: 1.1207x; 1.1207x over previous
import jax
import jax.numpy as jnp
from jax.experimental import pallas as pl
from jax.experimental.pallas import tpu as pltpu

_E = 8
_K = 2
_HB = 512


def _dense_body(x_ref, wr_ref, br_ref, w1_ref, b1_ref, w2_ref, b2_ref,
                out_ref, mask_ref, acc_ref):
    e = pl.program_id(0)
    hb = pl.program_id(1)
    nhb = pl.num_programs(1)
    T = x_ref.shape[0]

    @pl.when((e == 0) & (hb == 0))
    def _router():
        x = x_ref[...]
        logits = jnp.dot(x, wr_ref[...], preferred_element_type=jnp.float32)
        logits = logits + br_ref[...]
        col = jax.lax.broadcasted_iota(jnp.int32, (T, _E), 1)
        rank = jnp.zeros((T, _E), jnp.int32)
        for ep in range(_E):
            le = logits[:, ep:ep + 1]
            rank = rank + (le > logits).astype(jnp.int32)
            rank = rank + ((le == logits) & (ep < col)).astype(jnp.int32)
        mask_ref[...] = (rank < _K).astype(jnp.float32)
        acc_ref[...] = jnp.zeros_like(acc_ref)

    x = x_ref[...]
    col = jax.lax.broadcasted_iota(jnp.int32, (T, _E), 1)
    mask_e = jnp.sum(mask_ref[...] * (col == e).astype(jnp.float32),
                     axis=1, keepdims=True)

    h = jnp.maximum(
        jnp.dot(x, w1_ref[0], preferred_element_type=jnp.float32) + b1_ref[0],
        0.0)
    part = jnp.dot(h, w2_ref[0], preferred_element_type=jnp.float32)

    @pl.when(hb == 0)
    def _bias2():
        acc_ref[...] += mask_e * b2_ref[0]

    acc_ref[...] += mask_e * part

    @pl.when((e == _E - 1) & (hb == nhb - 1))
    def _emit():
        out_ref[...] = acc_ref[...] * (1.0 / _K)


def kernel(x, Wr, br, W1, b1, W2, b2):
    B_, S_, D_ = x.shape
    Eh, Dh, H = W1.shape
    xf = x.reshape(S_, D_)
    br2 = br.reshape(1, Eh)
    b1r = b1.reshape(Eh, 1, H)
    b2r = b2.reshape(Eh, 1, D_)

    out = pl.pallas_call(
        _dense_body,
        grid=(Eh, H // _HB),
        in_specs=[
            pl.BlockSpec((S_, D_), lambda e, hb: (0, 0)),
            pl.BlockSpec((D_, Eh), lambda e, hb: (0, 0)),
            pl.BlockSpec((1, Eh), lambda e, hb: (0, 0)),
            pl.BlockSpec((1, D_, _HB), lambda e, hb: (e, 0, hb)),
            pl.BlockSpec((1, 1, _HB), lambda e, hb: (e, 0, hb)),
            pl.BlockSpec((1, _HB, D_), lambda e, hb: (e, hb, 0)),
            pl.BlockSpec((1, 1, D_), lambda e, hb: (e, 0, 0)),
        ],
        out_specs=pl.BlockSpec((S_, D_), lambda e, hb: (0, 0)),
        out_shape=jax.ShapeDtypeStruct((S_, D_), jnp.float32),
        scratch_shapes=[
            pltpu.VMEM((S_, Eh), jnp.float32),
            pltpu.VMEM((S_, D_), jnp.float32),
        ],
        compiler_params=pltpu.CompilerParams(
            dimension_semantics=("arbitrary", "arbitrary")),
    )(xf, Wr, br2, W1, b1r, W2, b2r)
    return out.reshape(B_, S_, D_)
